# baseline (speedup 1.0000x reference)
import jax
import jax.numpy as jnp
from jax import lax
from jax.experimental import pallas as pl
from jax.experimental.pallas import tpu as pltpu
from jax.experimental.pallas import tpu_sc as plsc

ROWS = 128
COLS = 32768
K = 9
NUM_CORES = 2
NUM_SUBCORES = 16
NUM_WORKERS = NUM_CORES * NUM_SUBCORES
COLS_PER_WORKER = COLS // NUM_WORKERS
CHUNK = 256
NUM_CHUNKS = COLS_PER_WORKER // CHUNK
LANES = 16
LANE_GROUPS = CHUNK // LANES
ROW_UNROLL = 8


def _sc_body(x_hbm, kth_hbm, zero_hbm, buf, kth_buf, zbuf):
    cid = lax.axis_index("c")
    sid = lax.axis_index("s")
    wid = sid * NUM_CORES + cid
    col0 = wid * COLS_PER_WORKER

    @pl.when(wid == 0)
    def _():
        zbuf[...] = jnp.zeros((LANES,), jnp.int32)
        pltpu.sync_copy(zbuf, zero_hbm)

    for c in range(NUM_CHUNKS):
        base = col0 + c * CHUNK
        pltpu.sync_copy(x_hbm.at[:, pl.ds(base, CHUNK)], buf)

        def g_body(g, carry):
            g16 = pl.multiple_of(g * LANES, LANES)
            inf = jnp.full((LANES,), jnp.inf, jnp.float32)
            ms0 = (inf,) * K

            def row_blk(rb, ms):
                ms = list(ms)
                r0 = rb * ROW_UNROLL
                for rr in range(ROW_UNROLL):
                    v = buf[r0 + rr, pl.ds(g16, LANES)]
                    for i in range(K):
                        lo = jnp.minimum(ms[i], v)
                        v = jnp.maximum(ms[i], v)
                        ms[i] = lo
                return tuple(ms)

            ms = lax.fori_loop(0, ROWS // ROW_UNROLL, row_blk, ms0)
            kth_buf[pl.ds(g16, LANES)] = ms[K - 1]
            return carry

        lax.fori_loop(0, LANE_GROUPS, g_body, 0)
        pltpu.sync_copy(kth_buf, kth_hbm.at[pl.ds(base, CHUNK)])


_mesh = plsc.VectorSubcoreMesh(core_axis_name="c", subcore_axis_name="s")

_sc_call = pl.kernel(
    _sc_body,
    out_type=[
        jax.ShapeDtypeStruct((COLS,), jnp.float32),
        jax.ShapeDtypeStruct((LANES,), jnp.int32),
    ],
    mesh=_mesh,
    scratch_types=[
        pltpu.VMEM((ROWS, CHUNK), jnp.float32),
        pltpu.VMEM((CHUNK,), jnp.float32),
        pltpu.VMEM((LANES,), jnp.int32),
    ],
)


def kernel(x):
    kth_vals, zero = _sc_call(x)
    del kth_vals
    return zero[0]
